# baseline (speedup 1.0000x reference)
import functools

import jax
import jax.numpy as jnp
from jax import lax
from jax.experimental import pallas as pl
from jax.experimental.pallas import tpu as pltpu
from jax.experimental.pallas import tpu_sc as plsc

D_MODEL = 64
SCALE = float(D_MODEL) ** 0.5
NC, NS = 2, 16
NW = NC * NS
NBUF = 4


def _mesh():
    return plsc.VectorSubcoreMesh(core_axis_name="c", subcore_axis_name="s")


PANEL = 128


@functools.cache
def _make_transpose(v_rows):
    n_full = v_rows // PANEL
    n_iter = -(-n_full // NW)

    @functools.partial(
        pl.kernel,
        mesh=_mesh(),
        out_type=jax.ShapeDtypeStruct((v_rows // 2, 128), jnp.float32),
        scratch_types=(
            [pltpu.VMEM((D_MODEL, PANEL), jnp.float32)] * 2
            + [pltpu.VMEM((PANEL // 2, 128), jnp.float32)] * 2
            + [pltpu.SemaphoreType.DMA] * 4
        ),
        compiler_params=pltpu.CompilerParams(needs_layout_passes=False),
    )
    def tr(tt_hbm, tail_hbm, out_hbm, sb0, sb1, db0, db1, gi0, gi1, go0, go1):
        sbufs, dbufs = (sb0, sb1), (db0, db1)
        gis, gos = (gi0, gi1), (go0, go1)
        wid = lax.axis_index("s") * NC + lax.axis_index("c")
        iota = lax.iota(jnp.int32, 16)

        def unit_of(k):
            return jnp.minimum(k * NW + wid, n_full - 1)

        def in_start(k, p):
            pltpu.make_async_copy(
                tt_hbm.at[:, pl.ds(unit_of(k) * PANEL, PANEL)],
                sbufs[p], gis[p]).start()

        def in_wait(k, p):
            pltpu.make_async_copy(
                tt_hbm.at[:, pl.ds(unit_of(k) * PANEL, PANEL)],
                sbufs[p], gis[p]).wait()

        def out_start(k, p):
            pltpu.make_async_copy(
                dbufs[p],
                out_hbm.at[pl.ds(unit_of(k) * (PANEL // 2), PANEL // 2)],
                gos[p]).start()

        def out_wait(k, p):
            pltpu.make_async_copy(
                dbufs[p],
                out_hbm.at[pl.ds(unit_of(k) * (PANEL // 2), PANEL // 2)],
                gos[p]).wait()

        def transpose_panel(src, dst, n_pairs):
            @plsc.parallel_loop(0, n_pairs, unroll=8)
            def row(pp):
                for h in range(2):
                    col = jnp.full((16,), 2 * pp + h, jnp.int32)
                    for d0 in range(0, D_MODEL, 16):
                        vec = plsc.load_gather(src, [iota + d0, col])
                        dst[pp, pl.ds(h * D_MODEL + d0, 16)] = vec * SCALE

        in_start(0, 0)
        in_start(1, 1)

        in_wait(0, 0)
        transpose_panel(sb0, db0, PANEL // 2)
        out_start(0, 0)
        in_start(2, 0)

        in_wait(1, 1)
        transpose_panel(sb1, db1, PANEL // 2)
        out_start(1, 1)
        in_start(3, 1)

        def body(i, carry):
            for par in range(2):
                k = 2 * i + par
                in_wait(k, par)
                out_wait(k - 2, par)
                transpose_panel(sbufs[par], dbufs[par], PANEL // 2)
                out_start(k, par)
                in_start(k + 2, par)
            return carry
        lax.fori_loop(1, (n_iter - 3) // 2, body, 0)

        for k, fire in ((n_iter - 3, True), (n_iter - 2, False),
                        (n_iter - 1, False)):
            par = k % 2
            in_wait(k, par)
            out_wait(k - 2, par)
            transpose_panel(sbufs[par], dbufs[par], PANEL // 2)
            out_start(k, par)
            if fire:
                in_start(k + 2, par)

        out_wait(n_iter - 2, (n_iter - 2) % 2)
        out_wait(n_iter - 1, (n_iter - 1) % 2)

        @pl.when(wid == NW - 1)
        def _tail():
            pltpu.sync_copy(tail_hbm, sb0.at[:, pl.ds(0, 128)])
            transpose_panel(sb0, db0, 32)
            pltpu.sync_copy(db0.at[pl.ds(0, 32)],
                            out_hbm.at[pl.ds(n_full * (PANEL // 2), 32)])

    return tr


@functools.cache
def _make_gather(n_seq, seq_len):
    steps = n_seq // NW
    assert steps % NBUF == 0 and steps >= 2 * NBUF
    pad_rows = (seq_len + 7) // 8 * 8
    pad_cols = 128

    @functools.partial(
        pl.kernel,
        mesh=_mesh(),
        out_type=jax.ShapeDtypeStruct((n_seq, pad_rows, pad_cols), jnp.float32),
        scratch_types=(
            [pltpu.VMEM((steps, seq_len), jnp.int32)]
            + [pltpu.VMEM((seq_len, D_MODEL), jnp.float32)] * NBUF
            + [pltpu.SemaphoreType.DMA] * (2 * NBUF)
        ),
        compiler_params=pltpu.CompilerParams(use_tc_tiling_on_sc=False),
    )
    def emb(idx_hbm, table_hbm, out_hbm, idx_v, r0, r1, r2, r3,
            g0, g1, g2, g3, s0, s1, s2, s3):
        bufs = (r0, r1, r2, r3)
        gsems = (g0, g1, g2, g3)
        ssems = (s0, s1, s2, s3)
        wid = lax.axis_index("s") * NC + lax.axis_index("c")
        row0 = wid * steps

        pltpu.sync_copy(idx_hbm.at[pl.ds(row0, steps)], idx_v)

        def g_start(s, b):
            pltpu.make_async_copy(
                table_hbm.at[idx_v.at[s]], bufs[b], gsems[b]).start()

        def g_wait(s, b):
            pltpu.make_async_copy(
                table_hbm.at[idx_v.at[s]], bufs[b], gsems[b]).wait()

        def st_start(s, b):
            pltpu.make_async_copy(
                bufs[b],
                out_hbm.at[row0 + s, pl.ds(0, seq_len), pl.ds(0, D_MODEL)],
                ssems[b]).start()

        def st_wait(s, b):
            pltpu.make_async_copy(
                bufs[b],
                out_hbm.at[row0 + s, pl.ds(0, seq_len), pl.ds(0, D_MODEL)],
                ssems[b]).wait()

        g_start(0, 0)
        g_start(1, 1)

        g_start(2, 2)
        g_wait(0, 0)
        st_start(0, 0)

        g_start(3, 3)
        g_wait(1, 1)
        st_start(1, 1)

        def body(i, carry):
            for k in range(NBUF):
                s = 2 + i * NBUF + k
                b = (2 + k) % NBUF
                b2 = k % NBUF
                st_wait(s - 2, b2)
                g_start(s + 2, b2)
                g_wait(s, b)
                st_start(s, b)
            return carry
        lax.fori_loop(0, (steps - 4) // NBUF, body, 0)

        g_wait(steps - 2, 2)
        st_start(steps - 2, 2)

        g_wait(steps - 1, 3)
        st_start(steps - 1, 3)

        st_wait(steps - 4, 0)
        st_wait(steps - 3, 1)
        st_wait(steps - 2, 2)
        st_wait(steps - 1, 3)

    return emb


TC_BLK = 512


def _tc_transpose_body(x_ref, o_ref):
    xt = x_ref[...].T * SCALE
    o_ref[:, 0:D_MODEL] = xt
    o_ref[:, D_MODEL:2 * D_MODEL] = xt


@functools.cache
def _make_transpose_tc(v_rows):
    n_blk = -(-v_rows // TC_BLK)
    return pl.pallas_call(
        _tc_transpose_body,
        grid=(n_blk,),
        in_specs=[pl.BlockSpec((D_MODEL, TC_BLK), lambda i: (0, i))],
        out_specs=pl.BlockSpec((TC_BLK, 2 * D_MODEL), lambda i: (i, 0)),
        out_shape=jax.ShapeDtypeStruct((v_rows, 2 * D_MODEL), jnp.float32),
    )


def kernel(x, table):
    n_seq, seq_len = x.shape
    v_rows, d = table.shape
    tt = table.T
    ta = _make_transpose_tc(v_rows)(tt)
    tb = ta.reshape(2 * v_rows, d)
    padded = _make_gather(n_seq, seq_len)(x.astype(jnp.int32) * 2, tb)
    return padded[:, :seq_len, :D_MODEL]


# speedup vs baseline: 1.6772x; 1.6772x over previous
import functools

import jax
import jax.numpy as jnp
from jax import lax
from jax.experimental import pallas as pl
from jax.experimental.pallas import tpu as pltpu
from jax.experimental.pallas import tpu_sc as plsc

D_MODEL = 64
SCALE = float(D_MODEL) ** 0.5
NC, NS = 2, 16
NW = NC * NS
NBUF = 4


def _scale_chunk(rows, n_rows):
    def body(r, carry):
        for c in range(D_MODEL // 16):
            sl = (r, pl.ds(c * 16, 16))
            rows[sl] = rows[sl] * SCALE
        return carry
    lax.fori_loop(0, n_rows, body, 0, unroll=5)


@functools.cache
def _make_kernel(n_seq, seq_len):
    steps = n_seq // NW
    assert steps % NBUF == 0 and steps >= 2 * NBUF
    pad_rows = (seq_len + 7) // 8 * 8
    pad_cols = 128

    mesh = plsc.VectorSubcoreMesh(core_axis_name="c", subcore_axis_name="s")

    @functools.partial(
        pl.kernel,
        mesh=mesh,
        out_type=jax.ShapeDtypeStruct((n_seq, pad_rows, pad_cols), jnp.float32),
        scratch_types=(
            [pltpu.VMEM((steps, seq_len), jnp.int32)]
            + [pltpu.VMEM((seq_len, D_MODEL), jnp.float32)] * NBUF
            + [pltpu.SemaphoreType.DMA] * (2 * NBUF)
        ),
        compiler_params=pltpu.CompilerParams(use_tc_tiling_on_sc=False),
    )
    def emb(idx_hbm, table_hbm, out_hbm, idx_v, r0, r1, r2, r3,
            g0, g1, g2, g3, s0, s1, s2, s3):
        bufs = (r0, r1, r2, r3)
        gsems = (g0, g1, g2, g3)
        ssems = (s0, s1, s2, s3)
        wid = lax.axis_index("s") * NC + lax.axis_index("c")
        row0 = wid * steps

        pltpu.sync_copy(idx_hbm.at[pl.ds(row0, steps)], idx_v)

        def g_start(s, b):
            pltpu.make_async_copy(
                table_hbm.at[idx_v.at[s]], bufs[b], gsems[b]).start()

        def g_wait(s, b):
            pltpu.make_async_copy(
                table_hbm.at[idx_v.at[s]], bufs[b], gsems[b]).wait()

        def st_start(s, b):
            pltpu.make_async_copy(
                bufs[b],
                out_hbm.at[row0 + s, pl.ds(0, seq_len), pl.ds(0, D_MODEL)],
                ssems[b]).start()

        def st_wait(s, b):
            pltpu.make_async_copy(
                bufs[b],
                out_hbm.at[row0 + s, pl.ds(0, seq_len), pl.ds(0, D_MODEL)],
                ssems[b]).wait()

        g_start(0, 0)
        g_start(1, 1)

        g_start(2, 2)
        g_wait(0, 0)
        _scale_chunk(bufs[0], seq_len)
        st_start(0, 0)

        g_start(3, 3)
        g_wait(1, 1)
        _scale_chunk(bufs[1], seq_len)
        st_start(1, 1)

        def body(i, carry):
            for k in range(NBUF):
                s = 2 + i * NBUF + k
                b = (2 + k) % NBUF
                b2 = k % NBUF
                st_wait(s - 2, b2)
                g_start(s + 2, b2)
                g_wait(s, b)
                _scale_chunk(bufs[b], seq_len)
                st_start(s, b)
            return carry
        lax.fori_loop(0, (steps - 4) // NBUF, body, 0)

        g_wait(steps - 2, 2)
        _scale_chunk(bufs[2], seq_len)
        st_start(steps - 2, 2)

        g_wait(steps - 1, 3)
        _scale_chunk(bufs[3], seq_len)
        st_start(steps - 1, 3)

        st_wait(steps - 4, 0)
        st_wait(steps - 3, 1)
        st_wait(steps - 2, 2)
        st_wait(steps - 1, 3)

    return emb


def kernel(x, table):
    n_seq, seq_len = x.shape
    padded = _make_kernel(n_seq, seq_len)(x.astype(jnp.int32), table)
    return padded[:, :seq_len, :D_MODEL]
